# baseline (speedup 1.0000x reference)
import functools

import jax
import jax.numpy as jnp
from jax import lax
from jax.experimental import pallas as pl
from jax.experimental.pallas import tpu as pltpu
from jax.experimental.pallas import tpu_sc as plsc

N_NODES = 50000
D = 128
SEQ = 9
OUT_CH = 128

NW = 32
CHUNK = 128
LEAD = 3
M_BLK = 400

GROUP_SIZES = (12400, 12400, 12400, 12400, 400)


def _make_sc_body(cpw):
    def _sc_gather_body(x_hbm, idx_hbm, out_hbm, idx_v, buf_a, buf_b, buf_c,
                        gs_a, gs_b, gs_c):
        wid = lax.axis_index("s") * 2 + lax.axis_index("c")
        base_c = wid * cpw
        pltpu.sync_copy(idx_hbm.at[wid], idx_v)

        bufs, gsems = (buf_a, buf_b, buf_c), (gs_a, gs_b, gs_c)

        def g_start(c, b):
            pltpu.async_copy(x_hbm.at[idx_v.at[c]], bufs[b], gsems[b])

        def g_wait(c, b):
            pltpu.make_async_copy(x_hbm.at[idx_v.at[c]], bufs[b], gsems[b]).wait()

        def w_sync(c, b):
            pltpu.sync_copy(bufs[b], out_hbm.at[pl.ds((base_c + c) * CHUNK, CHUNK)])

        for b in range(min(LEAD, cpw)):
            g_start(b, b)

        n_main_blocks = max(0, cpw - LEAD) // LEAD

        def step(i, carry):
            c0 = i * LEAD
            for b in range(LEAD):
                g_wait(c0 + b, b)
                w_sync(c0 + b, b)
                g_start(c0 + b + LEAD, b)
            return carry

        lax.fori_loop(0, n_main_blocks, step, 0)

        for c in range(n_main_blocks * LEAD, cpw):
            b = c % LEAD
            g_wait(c, b)
            w_sync(c, b)
            if c + LEAD < cpw:
                g_start(c + LEAD, b)

    return _sc_gather_body


def _sc_gather(x, idx3d, cpw):
    mesh = plsc.VectorSubcoreMesh(core_axis_name="c", subcore_axis_name="s")
    k = functools.partial(
        pl.kernel,
        mesh=mesh,
        out_type=jax.ShapeDtypeStruct((NW * cpw * CHUNK, D), jnp.float32),
        scratch_types=[
            pltpu.VMEM((cpw, CHUNK), jnp.int32),
            pltpu.VMEM((CHUNK, D), jnp.float32),
            pltpu.VMEM((CHUNK, D), jnp.float32),
            pltpu.VMEM((CHUNK, D), jnp.float32),
            pltpu.SemaphoreType.DMA,
            pltpu.SemaphoreType.DMA,
            pltpu.SemaphoreType.DMA,
        ],
    )(_make_sc_body(cpw))
    return k(x, idx3d)


def _mm_body(*refs):
    g_refs, w_ref, b_ref = refs[:SEQ], refs[SEQ], refs[SEQ + 1]
    o_ref = refs[-1]
    acc = jnp.broadcast_to(b_ref[...], (M_BLK, OUT_CH))
    for s in range(SEQ):
        acc = acc + jnp.dot(g_refs[s][...], w_ref[s],
                            preferred_element_type=jnp.float32)
    o_ref[...] = acc


def _tc_matmul(gathered, w3, b2, nodes_g, base_blk, out_prev):
    n_mblks = nodes_g // M_BLK
    in_specs = [
        pl.BlockSpec((M_BLK, D),
                     functools.partial(lambda i, s, nb: (s * nb + i, 0), s=s, nb=n_mblks))
        for s in range(SEQ)
    ]
    in_specs.append(pl.BlockSpec((SEQ, D, OUT_CH), lambda i: (0, 0, 0)))
    in_specs.append(pl.BlockSpec((1, OUT_CH), lambda i: (0, 0)))
    args = [*([gathered] * SEQ), w3, b2]
    aliases = {}
    if out_prev is not None:
        in_specs.append(pl.BlockSpec(memory_space=pl.ANY))
        args.append(out_prev)
        aliases = {SEQ + 2: 0}
    return pl.pallas_call(
        _mm_body,
        grid=(n_mblks,),
        in_specs=in_specs,
        out_specs=pl.BlockSpec((M_BLK, OUT_CH),
                               functools.partial(lambda i, bb: (bb + i, 0), bb=base_blk)),
        out_shape=jax.ShapeDtypeStruct((N_NODES, OUT_CH), jnp.float32),
        input_output_aliases=aliases,
    )(*args)


def kernel(x, indices, W, b):
    w3 = W.reshape(SEQ, D, OUT_CH)
    b2 = b.reshape(1, OUT_CH)
    out = None
    node0 = 0
    for nodes_g in GROUP_SIZES:
        flat_g = nodes_g * SEQ
        cpw = -(-flat_g // (CHUNK * NW))
        flat_pad = NW * cpw * CHUNK
        idx_g = indices[node0:node0 + nodes_g].astype(jnp.int32).T.reshape(-1)
        idx_g = jnp.pad(idx_g, (0, flat_pad - flat_g)).reshape(NW, cpw, CHUNK)
        gathered = _sc_gather(x, idx_g, cpw)
        out = _tc_matmul(gathered, w3, b2, nodes_g, node0 // M_BLK, out)
        node0 += nodes_g
    return out


# speedup vs baseline: 3.4443x; 3.4443x over previous
import functools

import jax
import jax.numpy as jnp
from jax import lax
from jax.experimental import pallas as pl
from jax.experimental.pallas import tpu as pltpu
from jax.experimental.pallas import tpu_sc as plsc

N_NODES = 50000
D = 128
SEQ = 9
OUT_CH = 128

NW = 32
CHUNK = 128
LEAD = 3
M_BLK = 400

GROUP_SIZES = (10000, 10000, 10000, 10000, 10000)


def _make_sc_body(cpw):
    def _sc_gather_body(x_hbm, idx_hbm, out_hbm, idx_v, buf_a, buf_b, buf_c,
                        gs_a, gs_b, gs_c):
        wid = lax.axis_index("s") * 2 + lax.axis_index("c")
        base_c = wid * cpw
        pltpu.sync_copy(idx_hbm.at[wid], idx_v)

        bufs, gsems = (buf_a, buf_b, buf_c), (gs_a, gs_b, gs_c)

        def g_start(c, b):
            pltpu.async_copy(x_hbm.at[idx_v.at[c]], bufs[b], gsems[b])

        def g_wait(c, b):
            pltpu.make_async_copy(x_hbm.at[idx_v.at[c]], bufs[b], gsems[b]).wait()

        def w_sync(c, b):
            pltpu.sync_copy(bufs[b], out_hbm.at[pl.ds((base_c + c) * CHUNK, CHUNK)])

        for b in range(min(LEAD, cpw)):
            g_start(b, b)

        n_main_blocks = max(0, cpw - LEAD) // LEAD

        def step(i, carry):
            c0 = i * LEAD
            for b in range(LEAD):
                g_wait(c0 + b, b)
                w_sync(c0 + b, b)
                g_start(c0 + b + LEAD, b)
            return carry

        lax.fori_loop(0, n_main_blocks, step, 0)

        for c in range(n_main_blocks * LEAD, cpw):
            b = c % LEAD
            g_wait(c, b)
            w_sync(c, b)
            if c + LEAD < cpw:
                g_start(c + LEAD, b)

    return _sc_gather_body


def _sc_gather(x, idx3d, cpw):
    mesh = plsc.VectorSubcoreMesh(core_axis_name="c", subcore_axis_name="s")
    k = functools.partial(
        pl.kernel,
        mesh=mesh,
        out_type=jax.ShapeDtypeStruct((NW * cpw * CHUNK, D), jnp.float32),
        scratch_types=[
            pltpu.VMEM((cpw, CHUNK), jnp.int32),
            pltpu.VMEM((CHUNK, D), jnp.float32),
            pltpu.VMEM((CHUNK, D), jnp.float32),
            pltpu.VMEM((CHUNK, D), jnp.float32),
            pltpu.SemaphoreType.DMA,
            pltpu.SemaphoreType.DMA,
            pltpu.SemaphoreType.DMA,
        ],
    )(_make_sc_body(cpw))
    return k(x, idx3d)


def _mm_body(*refs):
    g_refs, w_ref, b_ref = refs[:SEQ], refs[SEQ], refs[SEQ + 1]
    o_ref = refs[-1]
    acc = jnp.broadcast_to(b_ref[...], (M_BLK, OUT_CH))
    for s in range(SEQ):
        acc = acc + jnp.dot(g_refs[s][...], w_ref[s],
                            preferred_element_type=jnp.float32)
    o_ref[...] = acc


def _tc_matmul(gathered, w3, b2, nodes_g, base_blk, out_prev):
    n_mblks = nodes_g // M_BLK
    in_specs = [
        pl.BlockSpec((M_BLK, D),
                     functools.partial(lambda i, s, nb: (s * nb + i, 0), s=s, nb=n_mblks))
        for s in range(SEQ)
    ]
    in_specs.append(pl.BlockSpec((SEQ, D, OUT_CH), lambda i: (0, 0, 0)))
    in_specs.append(pl.BlockSpec((1, OUT_CH), lambda i: (0, 0)))
    args = [*([gathered] * SEQ), w3, b2]
    aliases = {}
    if out_prev is not None:
        in_specs.append(pl.BlockSpec(memory_space=pl.ANY))
        args.append(out_prev)
        aliases = {SEQ + 2: 0}
    return pl.pallas_call(
        _mm_body,
        grid=(n_mblks,),
        in_specs=in_specs,
        out_specs=pl.BlockSpec((M_BLK, OUT_CH),
                               functools.partial(lambda i, bb: (bb + i, 0), bb=base_blk)),
        out_shape=jax.ShapeDtypeStruct((N_NODES, OUT_CH), jnp.float32),
        input_output_aliases=aliases,
    )(*args)


def kernel(x, indices, W, b):
    w3 = W.reshape(SEQ, D, OUT_CH)
    b2 = b.reshape(1, OUT_CH)
    out = None
    node0 = 0
    for nodes_g in GROUP_SIZES:
        flat_g = nodes_g * SEQ
        cpw = -(-flat_g // (CHUNK * NW))
        flat_pad = NW * cpw * CHUNK
        idx_g = indices[node0:node0 + nodes_g].astype(jnp.int32).T.reshape(-1)
        idx_g = jnp.pad(idx_g, (0, flat_pad - flat_g)).reshape(NW, cpw, CHUNK)
        gathered = _sc_gather(x, idx_g, cpw)
        out = _tc_matmul(gathered, w3, b2, nodes_g, node0 // M_BLK, out)
        node0 += nodes_g
    return out


# speedup vs baseline: 3.4472x; 1.0009x over previous
import functools

import jax
import jax.numpy as jnp
from jax import lax
from jax.experimental import pallas as pl
from jax.experimental.pallas import tpu as pltpu
from jax.experimental.pallas import tpu_sc as plsc

N_NODES = 50000
D = 128
SEQ = 9
OUT_CH = 128

NW = 32
CHUNK = 128
LEAD = 3
M_BLK = 400

GROUP_SIZES = (10000, 10000, 10000, 10000, 10000)


def _make_sc_body(cpw):
    def _sc_gather_body(x_hbm, idx_hbm, out_hbm, idx_v, b0, b1, b2, b3,
                        g0, g1, g2, g3, w0, w1, w2, w3):
        wid = lax.axis_index("s") * 2 + lax.axis_index("c")
        base_c = wid * cpw
        pltpu.sync_copy(idx_hbm.at[wid], idx_v)

        bufs, gsems, wsems = (b0, b1, b2, b3), (g0, g1, g2, g3), (w0, w1, w2, w3)

        def g_start(c, b):
            pltpu.async_copy(x_hbm.at[idx_v.at[c]], bufs[b], gsems[b])

        def g_wait(c, b):
            pltpu.make_async_copy(x_hbm.at[idx_v.at[c]], bufs[b], gsems[b]).wait()

        def out_slice(c):
            return out_hbm.at[pl.ds((base_c + c) * CHUNK, CHUNK)]

        def w_start(c, b):
            pltpu.async_copy(bufs[b], out_slice(c), wsems[b])

        def w_wait(c, b):
            pltpu.make_async_copy(bufs[b], out_slice(c), wsems[b]).wait()

        assert cpw > 8 and cpw % 2 == 0
        for b in range(4):
            g_start(b, b)
        g_wait(0, 0)
        w_start(0, 0)

        n_main = (cpw - 5) // 4 * 4

        def step(i, carry):
            c0 = 1 + i * 4
            for j in range(4):
                c = c0 + j
                b = (1 + j) % 4
                g_wait(c, b)
                w_start(c, b)
                wb = j % 4
                w_wait(c - 1, wb)
                g_start(c + 3, wb)
            return carry

        lax.fori_loop(0, n_main // 4, step, 0)

        for c in range(n_main + 1, cpw):
            b = c % 4
            g_wait(c, b)
            w_start(c, b)
            w_wait(c - 1, (c - 1) % 4)
            if c + 3 < cpw:
                g_start(c + 3, (c - 1) % 4)
        w_wait(cpw - 1, (cpw - 1) % 4)

    return _sc_gather_body


def _sc_gather(x, idx3d, cpw):
    mesh = plsc.VectorSubcoreMesh(core_axis_name="c", subcore_axis_name="s")
    k = functools.partial(
        pl.kernel,
        mesh=mesh,
        out_type=jax.ShapeDtypeStruct((NW * cpw * CHUNK, D), jnp.float32),
        scratch_types=[
            pltpu.VMEM((cpw, CHUNK), jnp.int32),
            pltpu.VMEM((CHUNK, D), jnp.float32),
            pltpu.VMEM((CHUNK, D), jnp.float32),
            pltpu.VMEM((CHUNK, D), jnp.float32),
            pltpu.VMEM((CHUNK, D), jnp.float32),
            pltpu.SemaphoreType.DMA,
            pltpu.SemaphoreType.DMA,
            pltpu.SemaphoreType.DMA,
            pltpu.SemaphoreType.DMA,
            pltpu.SemaphoreType.DMA,
            pltpu.SemaphoreType.DMA,
            pltpu.SemaphoreType.DMA,
            pltpu.SemaphoreType.DMA,
        ],
    )(_make_sc_body(cpw))
    return k(x, idx3d)


def _mm_body(*refs):
    g_refs, w_ref, b_ref = refs[:SEQ], refs[SEQ], refs[SEQ + 1]
    o_ref = refs[-1]
    acc = jnp.broadcast_to(b_ref[...], (M_BLK, OUT_CH))
    for s in range(SEQ):
        acc = acc + jnp.dot(g_refs[s][...], w_ref[s],
                            preferred_element_type=jnp.float32)
    o_ref[...] = acc


def _tc_matmul(gathered, w3, b2, nodes_g, base_blk, out_prev):
    n_mblks = nodes_g // M_BLK
    in_specs = [
        pl.BlockSpec((M_BLK, D),
                     functools.partial(lambda i, s, nb: (s * nb + i, 0), s=s, nb=n_mblks))
        for s in range(SEQ)
    ]
    in_specs.append(pl.BlockSpec((SEQ, D, OUT_CH), lambda i: (0, 0, 0)))
    in_specs.append(pl.BlockSpec((1, OUT_CH), lambda i: (0, 0)))
    args = [*([gathered] * SEQ), w3, b2]
    aliases = {}
    if out_prev is not None:
        in_specs.append(pl.BlockSpec(memory_space=pl.ANY))
        args.append(out_prev)
        aliases = {SEQ + 2: 0}
    return pl.pallas_call(
        _mm_body,
        grid=(n_mblks,),
        in_specs=in_specs,
        out_specs=pl.BlockSpec((M_BLK, OUT_CH),
                               functools.partial(lambda i, bb: (bb + i, 0), bb=base_blk)),
        out_shape=jax.ShapeDtypeStruct((N_NODES, OUT_CH), jnp.float32),
        input_output_aliases=aliases,
    )(*args)


def kernel(x, indices, W, b):
    w3 = W.reshape(SEQ, D, OUT_CH)
    b2 = b.reshape(1, OUT_CH)
    out = None
    node0 = 0
    for nodes_g in GROUP_SIZES:
        flat_g = nodes_g * SEQ
        cpw = -(-flat_g // (CHUNK * NW))
        flat_pad = NW * cpw * CHUNK
        idx_g = indices[node0:node0 + nodes_g].astype(jnp.int32).T.reshape(-1)
        idx_g = jnp.pad(idx_g, (0, flat_pad - flat_g)).reshape(NW, cpw, CHUNK)
        gathered = _sc_gather(x, idx_g, cpw)
        out = _tc_matmul(gathered, w3, b2, nodes_g, node0 // M_BLK, out)
        node0 += nodes_g
    return out
